# baseline (speedup 1.0000x reference)
import functools
import math

import jax
import jax.numpy as jnp
from jax import lax
from jax.experimental import pallas as pl
from jax.experimental.pallas import tpu as pltpu

N = 10000
E = 160000
IN_DIM = 256
H = 8
D = 32


def _matmul_bias_body(x_ref, w_ref, b_ref, o_ref):
    o_ref[...] = (
        jnp.dot(x_ref[...], w_ref[...], preferred_element_type=jnp.float32)
        + b_ref[...]
    )


def _matmul_bias(x, w, b, bm):
    m = x.shape[0]
    k = x.shape[1]
    p = w.shape[1]
    grid = (pl.cdiv(m, bm),)
    return pl.pallas_call(
        _matmul_bias_body,
        grid=grid,
        in_specs=[
            pl.BlockSpec((bm, k), lambda i: (i, 0)),
            pl.BlockSpec((k, p), lambda i: (0, 0)),
            pl.BlockSpec((1, p), lambda i: (0, 0)),
        ],
        out_specs=pl.BlockSpec((bm, p), lambda i: (i, 0)),
        out_shape=jax.ShapeDtypeStruct((m, p), jnp.float32),
    )(x, w, b.reshape(1, p))


def kernel(h, edge_index, e, Wq, bq, Wk, bk, Wv, bv, We, be):
    Q = _matmul_bias(h, Wq, bq, 512)
    K = _matmul_bias(h, Wk, bk, 512)
    V = _matmul_bias(h, Wv, bv, 512)
    pe = _matmul_bias(e, We, be, 512)

    src = edge_index[0]
    dst = edge_index[1]
    Qh = Q.reshape(N, H, D)
    Kh = K.reshape(N, H, D)
    Vh = V.reshape(N, H, D)
    peh = pe.reshape(E, H, D)
    score = (Kh[src] * Qh[dst]) / math.sqrt(D) * peh
    s_att = jnp.exp(jnp.clip(jnp.sum(score, axis=-1, keepdims=True), -5.0, 5.0))
    wV = jax.ops.segment_sum(Vh[src] * s_att, dst, num_segments=N)
    z = jax.ops.segment_sum(s_att, dst, num_segments=N)
    return (wV / (z + 1e-6)).reshape(N, H * D)


# speedup vs baseline: 1.1224x; 1.1224x over previous
import functools
import math

import jax
import jax.numpy as jnp
from jax import lax
from jax.experimental import pallas as pl
from jax.experimental.pallas import tpu as pltpu
from jax.experimental.pallas import tpu_sc as plsc

N = 10000
E = 160000
IN_DIM = 256
H = 8
D = 32

_NC = 2
_NS = 16
_NW = _NC * _NS
_CB = 200


def _gather_mul_body(k_hbm, q_hbm, src_hbm, dst_hbm, g_hbm,
                     sidx, didx, krows, qrows, sem0, sem1):
    wid = lax.axis_index("c") * _NS + lax.axis_index("s")
    per_w = E // _NW
    nchunks = per_w // _CB

    def chunk(j, carry):
        base = wid * per_w + j * _CB
        pltpu.sync_copy(src_hbm.at[pl.ds(base, _CB)], sidx)
        pltpu.sync_copy(dst_hbm.at[pl.ds(base, _CB)], didx)
        cp0 = pltpu.async_copy(k_hbm.at[sidx], krows, sem0)
        cp1 = pltpu.async_copy(q_hbm.at[didx], qrows, sem1)
        cp0.wait()
        cp1.wait()

        def row(i, c2):
            for cc in range(IN_DIM // 16):
                sl = pl.ds(cc * 16, 16)
                krows[i, sl] = krows[i, sl] * qrows[i, sl]
            return c2

        lax.fori_loop(0, _CB, row, 0)
        pltpu.sync_copy(krows, g_hbm.at[pl.ds(base, _CB)])
        return carry

    lax.fori_loop(0, nchunks, chunk, 0)


def _sc_gather_mul(k, q, src, dst):
    mesh = plsc.VectorSubcoreMesh(core_axis_name="c", subcore_axis_name="s")
    f = pl.kernel(
        _gather_mul_body,
        out_type=jax.ShapeDtypeStruct((E, IN_DIM), jnp.float32),
        mesh=mesh,
        scratch_types=[
            pltpu.VMEM((_CB,), jnp.int32),
            pltpu.VMEM((_CB,), jnp.int32),
            pltpu.VMEM((_CB, IN_DIM), jnp.float32),
            pltpu.VMEM((_CB, IN_DIM), jnp.float32),
            pltpu.SemaphoreType.DMA,
            pltpu.SemaphoreType.DMA,
        ],
    )
    return f(k, q, src, dst)


def _matmul_bias_body(x_ref, w_ref, b_ref, o_ref):
    o_ref[...] = (
        jnp.dot(x_ref[...], w_ref[...], preferred_element_type=jnp.float32)
        + b_ref[...]
    )


def _matmul_bias(x, w, b, bm):
    m = x.shape[0]
    k = x.shape[1]
    p = w.shape[1]
    grid = (pl.cdiv(m, bm),)
    return pl.pallas_call(
        _matmul_bias_body,
        grid=grid,
        in_specs=[
            pl.BlockSpec((bm, k), lambda i: (i, 0)),
            pl.BlockSpec((k, p), lambda i: (0, 0)),
            pl.BlockSpec((1, p), lambda i: (0, 0)),
        ],
        out_specs=pl.BlockSpec((bm, p), lambda i: (i, 0)),
        out_shape=jax.ShapeDtypeStruct((m, p), jnp.float32),
    )(x, w, b.reshape(1, p))


def _score_body(e_ref, g_ref, we_ref, be_ref, hsel_ref, o_ref):
    pe = (
        jnp.dot(e_ref[...], we_ref[...], preferred_element_type=jnp.float32)
        + be_ref[...]
    )
    t = pe * g_ref[...]
    s = jnp.dot(t, hsel_ref[...], preferred_element_type=jnp.float32)
    o_ref[...] = jnp.exp(jnp.clip(s * (1.0 / math.sqrt(D)), -5.0, 5.0))


def _tc_score(e, g, we, be, hsel, be_rows):
    grid = (E // be_rows,)
    return pl.pallas_call(
        _score_body,
        grid=grid,
        in_specs=[
            pl.BlockSpec((be_rows, IN_DIM), lambda i: (i, 0)),
            pl.BlockSpec((be_rows, IN_DIM), lambda i: (i, 0)),
            pl.BlockSpec((IN_DIM, H * D), lambda i: (0, 0)),
            pl.BlockSpec((1, H * D), lambda i: (0, 0)),
            pl.BlockSpec((H * D, 16), lambda i: (0, 0)),
        ],
        out_specs=pl.BlockSpec((be_rows, 16), lambda i: (i, 0)),
        out_shape=jax.ShapeDtypeStruct((E, 16), jnp.float32),
    )(e, g, we, be.reshape(1, H * D), hsel)


def kernel(h, edge_index, e, Wq, bq, Wk, bk, Wv, bv, We, be):
    Q = _matmul_bias(h, Wq, bq, 512)
    K = _matmul_bias(h, Wk, bk, 512)
    V = _matmul_bias(h, Wv, bv, 512)

    src = edge_index[0]
    dst = edge_index[1]
    g = _sc_gather_mul(K, Q, src, dst)
    hsel = jnp.concatenate(
        [jnp.repeat(jnp.eye(H, dtype=jnp.float32), D, axis=0),
         jnp.zeros((H * D, 16 - H), jnp.float32)], axis=1)
    s_att = _tc_score(e, g, We, be, hsel, 640)

    Vh = V.reshape(N, H, D)
    sa = s_att[:, :H].reshape(E, H, 1)
    wV = jax.ops.segment_sum(Vh[src] * sa, dst, num_segments=N)
    z = jax.ops.segment_sum(sa, dst, num_segments=N)
    return (wV / (z + 1e-6)).reshape(N, H * D)


# speedup vs baseline: 1.1554x; 1.0295x over previous
import functools
import math

import jax
import jax.numpy as jnp
from jax import lax
from jax.experimental import pallas as pl
from jax.experimental.pallas import tpu as pltpu
from jax.experimental.pallas import tpu_sc as plsc

N = 10000
E = 160000
IN_DIM = 256
H = 8
D = 32

_NC = 2
_NS = 16
_NW = _NC * _NS
_CB = 200


def _gather_mul_body(k_hbm, q_hbm, src_hbm, dst_hbm, g_hbm,
                     sidx, didx, krows, qrows, sem0, sem1):
    wid = lax.axis_index("c") * _NS + lax.axis_index("s")
    per_w = E // _NW
    nchunks = per_w // _CB

    def chunk(j, carry):
        base = wid * per_w + j * _CB
        pltpu.sync_copy(src_hbm.at[pl.ds(base, _CB)], sidx)
        pltpu.sync_copy(dst_hbm.at[pl.ds(base, _CB)], didx)
        cp0 = pltpu.async_copy(k_hbm.at[sidx], krows, sem0)
        cp1 = pltpu.async_copy(q_hbm.at[didx], qrows, sem1)
        cp0.wait()
        cp1.wait()

        def row(i, c2):
            for cc in range(IN_DIM // 16):
                sl = pl.ds(cc * 16, 16)
                krows[i, sl] = krows[i, sl] * qrows[i, sl]
            return c2

        lax.fori_loop(0, _CB, row, 0)
        pltpu.sync_copy(krows, g_hbm.at[pl.ds(base, _CB)])
        return carry

    lax.fori_loop(0, nchunks, chunk, 0)


def _sc_gather_mul(k, q, src, dst):
    mesh = plsc.VectorSubcoreMesh(core_axis_name="c", subcore_axis_name="s")
    f = pl.kernel(
        _gather_mul_body,
        out_type=jax.ShapeDtypeStruct((E, IN_DIM), jnp.float32),
        mesh=mesh,
        scratch_types=[
            pltpu.VMEM((_CB,), jnp.int32),
            pltpu.VMEM((_CB,), jnp.int32),
            pltpu.VMEM((_CB, IN_DIM), jnp.float32),
            pltpu.VMEM((_CB, IN_DIM), jnp.float32),
            pltpu.SemaphoreType.DMA,
            pltpu.SemaphoreType.DMA,
        ],
    )
    return f(k, q, src, dst)


def _weight_v_body(v_hbm, satt_hbm, src_hbm, ws_hbm,
                   sidx, vrows, srows, sem):
    wid = lax.axis_index("c") * _NS + lax.axis_index("s")
    per_w = E // _NW
    nchunks = per_w // _CB

    def chunk(j, carry):
        base = wid * per_w + j * _CB
        pltpu.sync_copy(src_hbm.at[pl.ds(base, _CB)], sidx)
        cp = pltpu.async_copy(v_hbm.at[sidx], vrows, sem)
        pltpu.sync_copy(satt_hbm.at[pl.ds(base, _CB)], srows)
        cp.wait()

        def row(i, c2):
            srow = srows[i, pl.ds(0, 16)]
            for hh in range(H):
                sv = jnp.full((16,), srow[hh], jnp.float32)
                for j2 in (2 * hh, 2 * hh + 1):
                    sl = pl.ds(j2 * 16, 16)
                    vrows[i, sl] = vrows[i, sl] * sv
            return c2

        lax.fori_loop(0, _CB, row, 0)
        pltpu.sync_copy(vrows, ws_hbm.at[pl.ds(base, _CB)])
        return carry

    lax.fori_loop(0, nchunks, chunk, 0)


def _sc_weight_v(v, satt, srcv):
    mesh = plsc.VectorSubcoreMesh(core_axis_name="c", subcore_axis_name="s")
    f = pl.kernel(
        _weight_v_body,
        out_type=jax.ShapeDtypeStruct((E, IN_DIM), jnp.float32),
        mesh=mesh,
        scratch_types=[
            pltpu.VMEM((_CB,), jnp.int32),
            pltpu.VMEM((_CB, IN_DIM), jnp.float32),
            pltpu.VMEM((_CB, 16), jnp.float32),
            pltpu.SemaphoreType.DMA,
        ],
    )
    return f(v, satt, srcv)


def _matmul_bias_body(x_ref, w_ref, b_ref, o_ref):
    o_ref[...] = (
        jnp.dot(x_ref[...], w_ref[...], preferred_element_type=jnp.float32)
        + b_ref[...]
    )


def _matmul_bias(x, w, b, bm):
    m = x.shape[0]
    k = x.shape[1]
    p = w.shape[1]
    grid = (pl.cdiv(m, bm),)
    return pl.pallas_call(
        _matmul_bias_body,
        grid=grid,
        in_specs=[
            pl.BlockSpec((bm, k), lambda i: (i, 0)),
            pl.BlockSpec((k, p), lambda i: (0, 0)),
            pl.BlockSpec((1, p), lambda i: (0, 0)),
        ],
        out_specs=pl.BlockSpec((bm, p), lambda i: (i, 0)),
        out_shape=jax.ShapeDtypeStruct((m, p), jnp.float32),
    )(x, w, b.reshape(1, p))


def _score_body(e_ref, g_ref, we_ref, be_ref, hsel_ref, o_ref):
    pe = (
        jnp.dot(e_ref[...], we_ref[...], preferred_element_type=jnp.float32)
        + be_ref[...]
    )
    t = pe * g_ref[...]
    s = jnp.dot(t, hsel_ref[...], preferred_element_type=jnp.float32)
    o_ref[...] = jnp.exp(jnp.clip(s * (1.0 / math.sqrt(D)), -5.0, 5.0))


def _tc_score(e, g, we, be, hsel, be_rows):
    grid = (E // be_rows,)
    return pl.pallas_call(
        _score_body,
        grid=grid,
        in_specs=[
            pl.BlockSpec((be_rows, IN_DIM), lambda i: (i, 0)),
            pl.BlockSpec((be_rows, IN_DIM), lambda i: (i, 0)),
            pl.BlockSpec((IN_DIM, H * D), lambda i: (0, 0)),
            pl.BlockSpec((1, H * D), lambda i: (0, 0)),
            pl.BlockSpec((H * D, 16), lambda i: (0, 0)),
        ],
        out_specs=pl.BlockSpec((be_rows, 16), lambda i: (i, 0)),
        out_shape=jax.ShapeDtypeStruct((E, 16), jnp.float32),
    )(e, g, we, be.reshape(1, H * D), hsel)


def kernel(h, edge_index, e, Wq, bq, Wk, bk, Wv, bv, We, be):
    Q = _matmul_bias(h, Wq, bq, 512)
    K = _matmul_bias(h, Wk, bk, 512)
    V = _matmul_bias(h, Wv, bv, 512)

    src = edge_index[0]
    dst = edge_index[1]
    g = _sc_gather_mul(K, Q, src, dst)
    hsel = jnp.concatenate(
        [jnp.repeat(jnp.eye(H, dtype=jnp.float32), D, axis=0),
         jnp.zeros((H * D, 16 - H), jnp.float32)], axis=1)
    s_att = _tc_score(e, g, We, be, hsel, 640)

    ws = _sc_weight_v(V, s_att, src)
    sa = s_att[:, :H].reshape(E, H, 1)
    wV = jax.ops.segment_sum(ws.reshape(E, H, D), dst, num_segments=N)
    z = jax.ops.segment_sum(sa, dst, num_segments=N)
    return (wV / (z + 1e-6)).reshape(N, H * D)
